# baseline (speedup 1.0000x reference)
import jax
import jax.numpy as jnp
from jax import lax
from jax.experimental import pallas as pl
from jax.experimental.pallas import tpu as pltpu
from jax.experimental.pallas import tpu_sc as plsc

_B = 16384
_F = 26
_V = 100000
_D = 32
_N = _B * _F
_NW = 32
_N_PER_W = _N // _NW
_IDX_COLS = 128
_IDX_ROWS = _N_PER_W // _IDX_COLS
_CHUNK = 1664
_N_CHUNKS = _N_PER_W // _CHUNK
_DMAS_PER_CHUNK = _CHUNK // _IDX_COLS


def _body(x_hbm, tab_hbm, out_hbm, idx_v, rows_v, gsem):
    wid = lax.axis_index("s") * 2 + lax.axis_index("c")
    base = wid * _N_PER_W

    pltpu.sync_copy(x_hbm.at[wid], idx_v)

    lane = lax.iota(jnp.int32, 16)

    def fix_row(r, carry):
        rowbase = base + r * _IDX_COLS
        for c in range(_IDX_COLS // 16):
            pos = lane + (rowbase + c * 16)
            f = pos % _F
            sl = pl.ds(c * 16, 16)
            idx_v[r, sl] = idx_v[r, sl] + f * _V
        return carry

    lax.fori_loop(0, _IDX_ROWS, fix_row, 0)

    def fire(g, buf):
        copies = []
        for j in range(_DMAS_PER_CHUNK):
            r = g * _DMAS_PER_CHUNK + j
            copies.append(
                pltpu.async_copy(
                    tab_hbm.at[idx_v.at[r]],
                    rows_v.at[buf, pl.ds(j * _IDX_COLS, _IDX_COLS)],
                    gsem,
                )
            )
        return copies

    pending = fire(0, 0)
    for g in range(_N_CHUNKS):
        for cp in pending:
            cp.wait()
        if g + 1 < _N_CHUNKS:
            nxt = fire(g + 1, (g + 1) % 2)
        else:
            nxt = []
        pltpu.sync_copy(
            rows_v.at[g % 2], out_hbm.at[pl.ds(base + g * _CHUNK, _CHUNK)]
        )
        pending = nxt


@jax.jit
def kernel(x_cat, tables):
    x3d = x_cat.reshape(_NW, _IDX_ROWS, _IDX_COLS)
    tab = tables.reshape(_F * _V, _D)
    mesh = plsc.VectorSubcoreMesh(core_axis_name="c", subcore_axis_name="s")
    out = pl.kernel(
        _body,
        mesh=mesh,
        out_type=jax.ShapeDtypeStruct((_N, _D), jnp.float32),
        scratch_types=[
            pltpu.VMEM((_IDX_ROWS, _IDX_COLS), jnp.int32),
            pltpu.VMEM((2, _CHUNK, _D), jnp.float32),
            pltpu.SemaphoreType.DMA,
        ],
        compiler_params=pltpu.CompilerParams(use_tc_tiling_on_sc=False),
    )(x3d, tab)
    return out.reshape(_B, _F, _D)


# speedup vs baseline: 3.8125x; 3.8125x over previous
import jax
import jax.numpy as jnp
from jax import lax
from jax.experimental import pallas as pl
from jax.experimental.pallas import tpu as pltpu
from jax.experimental.pallas import tpu_sc as plsc

_B = 16384
_F = 26
_V = 100000
_D = 32
_BC = 8192
_NB = _B // _BC
_GRP = _BC // 16


def _body(x_hbm, tab_hbm, out_hbm, row_v, idx_v, out_v):
    d = lax.axis_index("s") * 2 + lax.axis_index("c")

    def per_field(f, carry):
        pltpu.sync_copy(tab_hbm.at[f, d], row_v)

        def per_chunk(c, carry2):
            b0 = c * _BC
            pltpu.sync_copy(x_hbm.at[f, pl.ds(b0, _BC)], idx_v)

            def gather16(j, carry3):
                sl = pl.ds(j * 16, 16)
                iv = idx_v[sl]
                out_v[sl] = plsc.load_gather(row_v, [iv])
                return carry3

            lax.fori_loop(0, _GRP, gather16, 0)
            pltpu.sync_copy(out_v, out_hbm.at[f, d, pl.ds(b0, _BC)])
            return carry2

        lax.fori_loop(0, _NB, per_chunk, 0)
        return carry

    lax.fori_loop(0, _F, per_field, 0)


@jax.jit
def kernel(x_cat, tables):
    x_t = x_cat.T
    tab_t = jnp.transpose(tables, (0, 2, 1))
    mesh = plsc.VectorSubcoreMesh(core_axis_name="c", subcore_axis_name="s")
    out = pl.kernel(
        _body,
        mesh=mesh,
        out_type=jax.ShapeDtypeStruct((_F, _D, _B), jnp.float32),
        scratch_types=[
            pltpu.VMEM((_V,), jnp.float32),
            pltpu.VMEM((_BC,), jnp.int32),
            pltpu.VMEM((_BC,), jnp.float32),
        ],
        compiler_params=pltpu.CompilerParams(
            use_tc_tiling_on_sc=True, needs_layout_passes=False
        ),
    )(x_t, tab_t)
    return jnp.transpose(out, (2, 0, 1))
